# Initial kernel scaffold; baseline (speedup 1.0000x reference)
import functools

import jax
import jax.numpy as jnp
from jax import lax
from jax.experimental import pallas as pl
from jax.experimental.pallas import tpu as pltpu
from jax.experimental.pallas import tpu_sc as plsc

N = 10000
E = 320000
C = 128
NC = 2
NS = 16
NW = NC * NS
EPT = E // NW
B = 80
NWIN = EPT // B
LANES = 16
ROWS_PER_TILE = N // NS
ZROWS = 125
RB = 1000

NEG_SLOPE = 0.2


def _pre_body(x_ref, wl_ref, wr_ref, att_ref, xl_ref, xr_ref, ml_ref, mr_ref,
              mh_ref):
    i = pl.program_id(0)
    x = x_ref[...]
    dn = (((1,), (0,)), ((), ()))
    xl = lax.dot_general(x, wl_ref[...], dn,
                         precision=lax.Precision.HIGHEST,
                         preferred_element_type=jnp.float32)
    xr = lax.dot_general(x, wr_ref[...], dn,
                         precision=lax.Precision.HIGHEST,
                         preferred_element_type=jnp.float32)
    xl_ref[...] = xl
    xr_ref[...] = xr
    msql = jnp.max(jnp.sum(xl * xl, axis=1))
    msqr = jnp.max(jnp.sum(xr * xr, axis=1))

    @pl.when(i == 0)
    def _():
        ml_ref[...] = jnp.zeros_like(ml_ref)
        mr_ref[...] = jnp.zeros_like(mr_ref)

    ml_ref[...] = jnp.maximum(ml_ref[...], msql)
    mr_ref[...] = jnp.maximum(mr_ref[...], msqr)

    @pl.when(i == pl.num_programs(0) - 1)
    def _():
        a = att_ref[...]
        attn = jnp.sqrt(jnp.sum(a * a))
        mh_ref[...] = attn * (jnp.sqrt(ml_ref[...]) + jnp.sqrt(mr_ref[...]))


def _tc_pre(x, W_l, W_r, att2d):
    f32 = jnp.float32
    return pl.pallas_call(
        _pre_body,
        grid=(N // RB,),
        in_specs=[
            pl.BlockSpec((RB, C), lambda i: (i, 0)),
            pl.BlockSpec((C, C), lambda i: (0, 0)),
            pl.BlockSpec((C, C), lambda i: (0, 0)),
            pl.BlockSpec((1, C), lambda i: (0, 0)),
        ],
        out_specs=[
            pl.BlockSpec((RB, C), lambda i: (i, 0)),
            pl.BlockSpec((RB, C), lambda i: (i, 0)),
            pl.BlockSpec((8, 128), lambda i: (0, 0)),
            pl.BlockSpec((8, 128), lambda i: (0, 0)),
            pl.BlockSpec((8, 128), lambda i: (0, 0)),
        ],
        out_shape=[
            jax.ShapeDtypeStruct((N, C), f32),
            jax.ShapeDtypeStruct((N, C), f32),
            jax.ShapeDtypeStruct((8, 128), f32),
            jax.ShapeDtypeStruct((8, 128), f32),
            jax.ShapeDtypeStruct((8, 128), f32),
        ],
    )(x, W_l, W_r, att2d)


def _sc_edge_kernel(xl_hbm, xr_hbm, src_hbm, dst_hbm, att_hbm, mh_hbm,
                    outp_hbm, den_hbm, cnt_hbm,
                    src_v, dst_v, den_v, cnt_v, xl_v, xr_v, msg_v, ex_v,
                    att_v, mh_v, zbuf_v, acc_sh):
    c = lax.axis_index("c")
    s = lax.axis_index("s")
    wid = s * NC + c

    pltpu.sync_copy(src_hbm.at[wid], src_v)
    pltpu.sync_copy(dst_hbm.at[wid], dst_v)
    pltpu.sync_copy(att_hbm, att_v)
    pltpu.sync_copy(mh_hbm, mh_v)

    zero16 = jnp.zeros((LANES,), jnp.float32)

    @pl.loop(0, N, step=LANES)
    def _zero_dc(i):
        den_v[pl.ds(i, LANES)] = zero16
        cnt_v[pl.ds(i, LANES)] = zero16

    @pl.loop(0, ZROWS)
    def _zero_zb(r):
        for k in range(C // LANES):
            zbuf_v[r, pl.ds(k * LANES, LANES)] = zero16

    row0 = s * ROWS_PER_TILE
    for b in range(ROWS_PER_TILE // ZROWS):
        pltpu.sync_copy(zbuf_v, acc_sh.at[pl.ds(row0 + b * ZROWS, ZROWS)])
    plsc.subcore_barrier()

    att_chunks = [att_v[pl.ds(k * LANES, LANES)] for k in range(C // LANES)]
    mh16 = mh_v[...]

    @pl.loop(0, NWIN)
    def _window(w):
        pltpu.sync_copy(xl_hbm.at[src_v.at[w]], xl_v)
        pltpu.sync_copy(xr_hbm.at[dst_v.at[w]], xr_v)

        @pl.loop(0, B)
        def _logit(e):
            acc = zero16
            for k in range(C // LANES):
                a = xl_v[e, pl.ds(k * LANES, LANES)]
                bb = xr_v[e, pl.ds(k * LANES, LANES)]
                v = a + bb
                lr = jnp.maximum(v, NEG_SLOPE * v)
                acc = acc + lr * att_chunks[k]
            ex_v[e] = jnp.sum(acc)

        for j in range(B // LANES):
            lg = ex_v[pl.ds(j * LANES, LANES)]
            ex_v[pl.ds(j * LANES, LANES)] = jnp.exp(lg - mh16)

        @pl.loop(0, B)
        def _scale(e):
            ex = ex_v[e]
            d = dst_v[w, e]
            den_v[d] = den_v[d] + ex
            cnt_v[d] = cnt_v[d] + 1.0
            exb = jnp.full((LANES,), ex)
            for k in range(C // LANES):
                msg_v[e, pl.ds(k * LANES, LANES)] = (
                    xl_v[e, pl.ds(k * LANES, LANES)] * exb)

        pltpu.sync_copy(msg_v, acc_sh.at[dst_v.at[w]], add=True)

    plsc.subcore_barrier()
    pltpu.sync_copy(den_v, den_hbm.at[wid])
    pltpu.sync_copy(cnt_v, cnt_hbm.at[wid])
    for b in range(ROWS_PER_TILE // ZROWS):
        r = row0 + b * ZROWS
        pltpu.sync_copy(acc_sh.at[pl.ds(r, ZROWS)],
                        outp_hbm.at[c, pl.ds(r, ZROWS)])


def _sc_edge(xl, xr, src_w, dst_w, att, mh16):
    f32 = jnp.float32
    mesh = plsc.VectorSubcoreMesh(core_axis_name="c", subcore_axis_name="s")
    run = pl.kernel(
        _sc_edge_kernel,
        out_type=(
            jax.ShapeDtypeStruct((NC, N, C), f32),
            jax.ShapeDtypeStruct((NW, N), f32),
            jax.ShapeDtypeStruct((NW, N), f32),
        ),
        mesh=mesh,
        scratch_types=[
            pltpu.VMEM((NWIN, B), jnp.int32),
            pltpu.VMEM((NWIN, B), jnp.int32),
            pltpu.VMEM((N,), f32),
            pltpu.VMEM((N,), f32),
            pltpu.VMEM((B, C), f32),
            pltpu.VMEM((B, C), f32),
            pltpu.VMEM((B, C), f32),
            pltpu.VMEM((B,), f32),
            pltpu.VMEM((C,), f32),
            pltpu.VMEM((LANES,), f32),
            pltpu.VMEM((ZROWS, C), f32),
            pltpu.VMEM_SHARED((N, C), f32),
        ],
    )
    return run(xl, xr, src_w, dst_w, att, mh16)


def _post_body(outp_ref, den_ref, cnt_ref, bias_ref, o_ref):
    acc = outp_ref[0] + outp_ref[1]
    den = jnp.sum(den_ref[...], axis=0)
    cnt = jnp.sum(cnt_ref[...], axis=0)
    ok = jnp.logical_and(cnt > 0.0, den > 0.0)
    scale = jnp.where(ok, 1.0 / (den * jnp.maximum(cnt, 1.0)), 0.0)
    o_ref[...] = acc * scale[:, None] + bias_ref[...]


def _tc_post(outp, den, cnt, bias2d):
    return pl.pallas_call(
        _post_body,
        grid=(N // RB,),
        in_specs=[
            pl.BlockSpec((NC, RB, C), lambda i: (0, i, 0)),
            pl.BlockSpec((NW, RB), lambda i: (0, i)),
            pl.BlockSpec((NW, RB), lambda i: (0, i)),
            pl.BlockSpec((1, C), lambda i: (0, 0)),
        ],
        out_specs=pl.BlockSpec((RB, C), lambda i: (i, 0)),
        out_shape=jax.ShapeDtypeStruct((N, C), jnp.float32),
    )(outp, den, cnt, bias2d)


def kernel(x, edge_index, W_l, W_r, att, bias):
    xl, xr, _, _, mh = _tc_pre(x, W_l, W_r, att.reshape(1, C))
    mh16 = mh[0, :LANES]
    ei = edge_index.reshape(2, NW, NWIN, B)
    outp, den, cnt = _sc_edge(xl, xr, ei[0], ei[1], att, mh16)
    return _tc_post(outp, den, cnt, bias.reshape(1, C))


# speedup vs baseline: 6.2147x; 6.2147x over previous
import dataclasses
import functools

import jax
import jax.numpy as jnp
from jax import lax
from jax.experimental import pallas as pl
from jax.experimental.pallas import tpu as pltpu
from jax.experimental.pallas import tpu_sc as plsc

N = 10000
E = 320000
C = 128
NC = 2
NS = 16
NW = NC * NS
EPT = E // NW
B = 80
NWIN = EPT // B
LANES = 16
NP = 10240
RB = 1000

NEG_SLOPE = 0.2


def _pre_body(x_ref, wl_ref, wr_ref, att_ref, xl_ref, xr_ref, ml_ref, mr_ref,
              mh_ref):
    i = pl.program_id(0)
    x = x_ref[...]
    dn = (((1,), (0,)), ((), ()))
    xl = lax.dot_general(x, wl_ref[...], dn,
                         precision=lax.Precision.HIGHEST,
                         preferred_element_type=jnp.float32)
    xr = lax.dot_general(x, wr_ref[...], dn,
                         precision=lax.Precision.HIGHEST,
                         preferred_element_type=jnp.float32)
    xl_ref[...] = xl
    xr_ref[...] = xr
    msql = jnp.max(jnp.sum(xl * xl, axis=1))
    msqr = jnp.max(jnp.sum(xr * xr, axis=1))

    @pl.when(i == 0)
    def _():
        ml_ref[...] = jnp.zeros_like(ml_ref)
        mr_ref[...] = jnp.zeros_like(mr_ref)

    ml_ref[...] = jnp.maximum(ml_ref[...], msql)
    mr_ref[...] = jnp.maximum(mr_ref[...], msqr)

    @pl.when(i == pl.num_programs(0) - 1)
    def _():
        a = att_ref[...]
        attn = jnp.sqrt(jnp.sum(a * a))
        mh_ref[...] = attn * (jnp.sqrt(ml_ref[...]) + jnp.sqrt(mr_ref[...]))


def _tc_pre(x, W_l, W_r, att2d):
    f32 = jnp.float32
    return pl.pallas_call(
        _pre_body,
        grid=(N // RB,),
        in_specs=[
            pl.BlockSpec((RB, C), lambda i: (i, 0)),
            pl.BlockSpec((C, C), lambda i: (0, 0)),
            pl.BlockSpec((C, C), lambda i: (0, 0)),
            pl.BlockSpec((1, C), lambda i: (0, 0)),
        ],
        out_specs=[
            pl.BlockSpec((RB, C), lambda i: (i, 0)),
            pl.BlockSpec((RB, C), lambda i: (i, 0)),
            pl.BlockSpec((8, 128), lambda i: (0, 0)),
            pl.BlockSpec((8, 128), lambda i: (0, 0)),
            pl.BlockSpec((8, 128), lambda i: (0, 0)),
        ],
        out_shape=[
            jax.ShapeDtypeStruct((N, C), f32),
            jax.ShapeDtypeStruct((N, C), f32),
            jax.ShapeDtypeStruct((8, 128), f32),
            jax.ShapeDtypeStruct((8, 128), f32),
            jax.ShapeDtypeStruct((8, 128), f32),
        ],
    )(x, W_l, W_r, att2d)


HALF = 5120
TROWS = HALF // NS
ZROWS = 16
ZCHUNKS = TROWS // ZROWS
DH = NP // C


def _sc_edge_kernel(xl_hbm, xr_hbm, src_hbm, dst_hbm, att_hbm, mh_hbm,
                    outp_hbm, den_hbm, cnt_hbm,
                    srcw_v, dstw_v, xl_v, xr_v, ex_all, exz_v, cdst_v,
                    den_v, cnt_v, att_v, mh_v, zbuf_v, acc_sh):
    c = lax.axis_index("c")
    s = lax.axis_index("s")
    wid = s * NC + c

    pltpu.sync_copy(att_hbm, att_v)
    pltpu.sync_copy(mh_hbm, mh_v)

    zero16f = jnp.zeros((LANES,), jnp.float32)

    @pl.loop(0, ZROWS)
    def _zero_zb(r):
        for k in range(C // LANES):
            zbuf_v[r, pl.ds(k * LANES, LANES)] = zero16f

    @pl.loop(0, DH)
    def _zero_dc(r):
        for k in range(C // LANES):
            den_v[r, pl.ds(k * LANES, LANES)] = zero16f
            cnt_v[r, pl.ds(k * LANES, LANES)] = zero16f

    att_chunks = [att_v[pl.ds(k * LANES, LANES)] for k in range(C // LANES)]
    mh16 = mh_v[...]
    last_lane = lax.iota(jnp.int32, LANES) == (LANES - 1)
    ones16 = jnp.ones((LANES,), jnp.float32)
    row0 = s * TROWS

    def zero_acc():
        for b in range(ZCHUNKS):
            pltpu.sync_copy(zbuf_v, acc_sh.at[pl.ds(row0 + b * ZROWS, ZROWS)])

    def drain_acc(p):
        for b in range(ZCHUNKS):
            r = row0 + b * ZROWS
            pltpu.sync_copy(acc_sh.at[pl.ds(r, ZROWS)],
                            outp_hbm.at[c, p, pl.ds(r, ZROWS)])

    def route(dst16, ex16, base):
        idxm = dst16 - base
        neg = idxm < 0
        high = idxm >= HALF
        valid = jnp.logical_and(jnp.logical_not(neg), jnp.logical_not(high))
        cidx = idxm + jnp.where(neg, HALF, 0) - jnp.where(high, HALF, 0)
        exz = jnp.where(valid, ex16, 0.0)
        return cidx, exz

    def scale_and_scatter(w, base):
        for j in range(B // LANES):
            ex16 = ex_all[w, pl.ds(j * LANES, LANES)]
            dst16 = dstw_v[pl.ds(j * LANES, LANES)]
            cidx, exz = route(dst16, ex16, base)
            cdst_v[pl.ds(j * LANES, LANES)] = cidx
            exz_v[pl.ds(j * LANES, LANES)] = exz

        @pl.loop(0, B)
        def _scale(e):
            exb = plsc.load_gather(exz_v, [jnp.full((LANES,), e, jnp.int32)])
            for k in range(C // LANES):
                xl_v[e, pl.ds(k * LANES, LANES)] = (
                    xl_v[e, pl.ds(k * LANES, LANES)] * exb)

        pltpu.sync_copy(xl_v, acc_sh.at[cdst_v], add=True)

    zero_acc()
    plsc.subcore_barrier()

    ebase = wid * EPT

    @pl.loop(0, NWIN)
    def _window0(w):
        pltpu.sync_copy(src_hbm.at[pl.ds(ebase + w * B, B)], srcw_v)
        pltpu.sync_copy(dst_hbm.at[pl.ds(ebase + w * B, B)], dstw_v)
        pltpu.sync_copy(xl_hbm.at[srcw_v], xl_v)
        pltpu.sync_copy(xr_hbm.at[dstw_v], xr_v)

        @pl.loop(0, B)
        def _logit(e):
            acc = zero16f
            for k in range(C // LANES):
                a = xl_v[e, pl.ds(k * LANES, LANES)]
                bb = xr_v[e, pl.ds(k * LANES, LANES)]
                v = a + bb
                lr = jnp.maximum(v, NEG_SLOPE * v)
                acc = acc + lr * att_chunks[k]
            cum = plsc.cumsum(acc)
            plsc.store_scatter(ex_all,
                               [jnp.full((LANES,), w, jnp.int32),
                                jnp.full((LANES,), e, jnp.int32)],
                               cum, mask=last_lane)

        for j in range(B // LANES):
            lg = ex_all[w, pl.ds(j * LANES, LANES)]
            ex = jnp.exp(lg - mh16)
            ex_all[w, pl.ds(j * LANES, LANES)] = ex
            dst16 = dstw_v[pl.ds(j * LANES, LANES)]
            hi = lax.shift_right_logical(dst16, 7)
            lo = lax.bitwise_and(dst16, 127)
            plsc.addupdate_scatter(den_v, [hi, lo], ex)
            plsc.addupdate_scatter(cnt_v, [hi, lo], ones16)

        scale_and_scatter(w, 0)

    plsc.subcore_barrier()
    drain_acc(0)
    zero_acc()
    plsc.subcore_barrier()

    @pl.loop(0, NWIN)
    def _window1(w):
        pltpu.sync_copy(src_hbm.at[pl.ds(ebase + w * B, B)], srcw_v)
        pltpu.sync_copy(dst_hbm.at[pl.ds(ebase + w * B, B)], dstw_v)
        pltpu.sync_copy(xl_hbm.at[srcw_v], xl_v)
        scale_and_scatter(w, HALF)

    plsc.subcore_barrier()
    drain_acc(1)
    pltpu.sync_copy(den_v, den_hbm.at[wid])
    pltpu.sync_copy(cnt_v, cnt_hbm.at[wid])


def _sc_edge(xl, xr, src_w, dst_w, att, mh16):
    f32 = jnp.float32
    mesh = plsc.VectorSubcoreMesh(core_axis_name="c", subcore_axis_name="s")
    cp = pltpu.CompilerParams()
    if "needs_layout_passes" in pltpu.CompilerParams.__dataclass_fields__:
        cp = dataclasses.replace(cp, needs_layout_passes=False)
    run = pl.kernel(
        _sc_edge_kernel,
        out_type=(
            jax.ShapeDtypeStruct((NC, 2, HALF, C), f32),
            jax.ShapeDtypeStruct((NW, DH, C), f32),
            jax.ShapeDtypeStruct((NW, DH, C), f32),
        ),
        mesh=mesh,
        scratch_types=[
            pltpu.VMEM((B,), jnp.int32),
            pltpu.VMEM((B,), jnp.int32),
            pltpu.VMEM((B, C), f32),
            pltpu.VMEM((B, C), f32),
            pltpu.VMEM((NWIN, B), f32),
            pltpu.VMEM((B,), f32),
            pltpu.VMEM((B,), jnp.int32),
            pltpu.VMEM((DH, C), f32),
            pltpu.VMEM((DH, C), f32),
            pltpu.VMEM((C,), f32),
            pltpu.VMEM((LANES,), f32),
            pltpu.VMEM((ZROWS, C), f32),
            pltpu.VMEM_SHARED((HALF, C), f32),
        ],
        compiler_params=cp,
    )
    return run(xl, xr, src_w, dst_w, att, mh16)


def _post_body(outp_ref, den_ref, cnt_ref, bias_ref, o_ref):
    acc = (outp_ref[0].reshape(2 * HALF, C)[:N]
           + outp_ref[1].reshape(2 * HALF, C)[:N])
    den = jnp.sum(den_ref[...], axis=0)[:N]
    cnt = jnp.sum(cnt_ref[...], axis=0)[:N]
    ok = jnp.logical_and(cnt > 0.0, den > 0.0)
    scale = jnp.where(ok, 1.0 / (den * jnp.maximum(cnt, 1.0)), 0.0)
    o_ref[...] = acc * scale[:, None] + bias_ref[...]


def _tc_post(outp, den, cnt, bias2d):
    return pl.pallas_call(
        _post_body,
        out_shape=jax.ShapeDtypeStruct((N, C), jnp.float32),
    )(outp, den, cnt, bias2d)


def kernel(x, edge_index, W_l, W_r, att, bias):
    xl, xr, _, _, mh = _tc_pre(x, W_l, W_r, att.reshape(1, C))
    mh16 = mh[0, :LANES]
    outp, den, cnt = _sc_edge(xl, xr, edge_index[0], edge_index[1], att, mh16)
    den2 = den.reshape(NW, NP)[:, :N]
    cnt2 = cnt.reshape(NW, NP)[:, :N]
    return _tc_post(outp, den2, cnt2, bias.reshape(1, C))


# speedup vs baseline: 7.6510x; 1.2311x over previous
import dataclasses
import functools

import jax
import jax.numpy as jnp
from jax import lax
from jax.experimental import pallas as pl
from jax.experimental.pallas import tpu as pltpu
from jax.experimental.pallas import tpu_sc as plsc

N = 10000
E = 320000
C = 128
NC = 2
NS = 16
NW = NC * NS
EPT = E // NW
B = 80
NWIN = EPT // B
LANES = 16
NP = 10240
RB = 1000

NEG_SLOPE = 0.2


def _pre_body(x_ref, wl_ref, wr_ref, att_ref, xl_ref, xr_ref, ml_ref, mr_ref,
              mh_ref):
    i = pl.program_id(0)
    x = x_ref[...]
    dn = (((1,), (0,)), ((), ()))
    xl = lax.dot_general(x, wl_ref[...], dn,
                         precision=lax.Precision.HIGHEST,
                         preferred_element_type=jnp.float32)
    xr = lax.dot_general(x, wr_ref[...], dn,
                         precision=lax.Precision.HIGHEST,
                         preferred_element_type=jnp.float32)
    xl_ref[...] = xl
    xr_ref[...] = xr
    msql = jnp.max(jnp.sum(xl * xl, axis=1))
    msqr = jnp.max(jnp.sum(xr * xr, axis=1))

    @pl.when(i == 0)
    def _():
        ml_ref[...] = jnp.zeros_like(ml_ref)
        mr_ref[...] = jnp.zeros_like(mr_ref)

    ml_ref[...] = jnp.maximum(ml_ref[...], msql)
    mr_ref[...] = jnp.maximum(mr_ref[...], msqr)

    @pl.when(i == pl.num_programs(0) - 1)
    def _():
        a = att_ref[...]
        attn = jnp.sqrt(jnp.sum(a * a))
        mh_ref[...] = attn * (jnp.sqrt(ml_ref[...]) + jnp.sqrt(mr_ref[...]))


def _tc_pre(x, W_l, W_r, att2d):
    f32 = jnp.float32
    return pl.pallas_call(
        _pre_body,
        grid=(N // RB,),
        in_specs=[
            pl.BlockSpec((RB, C), lambda i: (i, 0)),
            pl.BlockSpec((C, C), lambda i: (0, 0)),
            pl.BlockSpec((C, C), lambda i: (0, 0)),
            pl.BlockSpec((1, C), lambda i: (0, 0)),
        ],
        out_specs=[
            pl.BlockSpec((RB, C), lambda i: (i, 0)),
            pl.BlockSpec((RB, C), lambda i: (i, 0)),
            pl.BlockSpec((8, 128), lambda i: (0, 0)),
            pl.BlockSpec((8, 128), lambda i: (0, 0)),
            pl.BlockSpec((8, 128), lambda i: (0, 0)),
        ],
        out_shape=[
            jax.ShapeDtypeStruct((N, C), f32),
            jax.ShapeDtypeStruct((N, C), f32),
            jax.ShapeDtypeStruct((8, 128), f32),
            jax.ShapeDtypeStruct((8, 128), f32),
            jax.ShapeDtypeStruct((8, 128), f32),
        ],
    )(x, W_l, W_r, att2d)


HALF = 5120
TROWS = HALF // NS
ZROWS = 16
ZCHUNKS = TROWS // ZROWS
DH = NP // C


def _sc_edge_kernel(xl_hbm, xr_hbm, src_hbm, dst_hbm, att_hbm, mh_hbm,
                    outp_hbm, den_hbm, cnt_hbm,
                    srcw_v, dstw_v, xl_v, xr_v, ex_all, exz_v, cdst_v,
                    den_v, cnt_v, att_v, mh_v, zbuf_v, acc_sh):
    c = lax.axis_index("c")
    s = lax.axis_index("s")
    wid = s * NC + c

    pltpu.sync_copy(att_hbm, att_v)
    pltpu.sync_copy(mh_hbm, mh_v)

    zero16f = jnp.zeros((LANES,), jnp.float32)

    @pl.loop(0, ZROWS)
    def _zero_zb(r):
        for k in range(C // LANES):
            zbuf_v[r, pl.ds(k * LANES, LANES)] = zero16f

    @pl.loop(0, DH)
    def _zero_dc(r):
        for k in range(C // LANES):
            den_v[r, pl.ds(k * LANES, LANES)] = zero16f
            cnt_v[r, pl.ds(k * LANES, LANES)] = zero16f

    att_chunks = [att_v[pl.ds(k * LANES, LANES)] for k in range(C // LANES)]
    mh16 = mh_v[...]
    last_lane = lax.iota(jnp.int32, LANES) == (LANES - 1)
    ones16 = jnp.ones((LANES,), jnp.float32)
    row0 = s * TROWS

    def zero_acc():
        for b in range(ZCHUNKS):
            pltpu.sync_copy(zbuf_v, acc_sh.at[pl.ds(row0 + b * ZROWS, ZROWS)])

    def drain_acc(p):
        for b in range(ZCHUNKS):
            r = row0 + b * ZROWS
            pltpu.sync_copy(acc_sh.at[pl.ds(r, ZROWS)],
                            outp_hbm.at[c, p, pl.ds(r, ZROWS)])

    def route(dst16, ex16, base):
        idxm = dst16 - base
        neg = idxm < 0
        high = idxm >= HALF
        valid = jnp.logical_and(jnp.logical_not(neg), jnp.logical_not(high))
        cidx = idxm + jnp.where(neg, HALF, 0) - jnp.where(high, HALF, 0)
        exz = jnp.where(valid, ex16, 0.0)
        return cidx, exz

    def scale_and_scatter(w, base):
        for j in range(B // LANES):
            ex16 = ex_all[w, pl.ds(j * LANES, LANES)]
            dst16 = dstw_v[pl.ds(j * LANES, LANES)]
            cidx, exz = route(dst16, ex16, base)
            cdst_v[pl.ds(j * LANES, LANES)] = cidx
            exz_v[pl.ds(j * LANES, LANES)] = exz

        @plsc.parallel_loop(0, B, unroll=4)
        def _scale(e):
            exb = plsc.load_gather(exz_v, [jnp.full((LANES,), e, jnp.int32)])
            for k in range(C // LANES):
                xl_v[e, pl.ds(k * LANES, LANES)] = (
                    xl_v[e, pl.ds(k * LANES, LANES)] * exb)

        pltpu.sync_copy(xl_v, acc_sh.at[cdst_v], add=True)

    zero_acc()
    plsc.subcore_barrier()

    ebase = wid * EPT

    @pl.loop(0, NWIN)
    def _window0(w):
        pltpu.sync_copy(src_hbm.at[pl.ds(ebase + w * B, B)], srcw_v)
        pltpu.sync_copy(dst_hbm.at[pl.ds(ebase + w * B, B)], dstw_v)
        pltpu.sync_copy(xl_hbm.at[srcw_v], xl_v)
        pltpu.sync_copy(xr_hbm.at[dstw_v], xr_v)

        @plsc.parallel_loop(0, B, unroll=2)
        def _logit(e):
            acc = zero16f
            for k in range(C // LANES):
                a = xl_v[e, pl.ds(k * LANES, LANES)]
                bb = xr_v[e, pl.ds(k * LANES, LANES)]
                v = a + bb
                lr = jnp.maximum(v, NEG_SLOPE * v)
                acc = acc + lr * att_chunks[k]
            cum = plsc.cumsum(acc)
            plsc.store_scatter(ex_all,
                               [jnp.full((LANES,), w, jnp.int32),
                                jnp.full((LANES,), e, jnp.int32)],
                               cum, mask=last_lane)

        for j in range(B // LANES):
            lg = ex_all[w, pl.ds(j * LANES, LANES)]
            ex = jnp.exp(lg - mh16)
            ex_all[w, pl.ds(j * LANES, LANES)] = ex
            dst16 = dstw_v[pl.ds(j * LANES, LANES)]
            hi = lax.shift_right_logical(dst16, 7)
            lo = lax.bitwise_and(dst16, 127)
            plsc.addupdate_scatter(den_v, [hi, lo], ex)
            plsc.addupdate_scatter(cnt_v, [hi, lo], ones16)

        scale_and_scatter(w, 0)

    plsc.subcore_barrier()
    drain_acc(0)
    zero_acc()
    plsc.subcore_barrier()

    @pl.loop(0, NWIN)
    def _window1(w):
        pltpu.sync_copy(src_hbm.at[pl.ds(ebase + w * B, B)], srcw_v)
        pltpu.sync_copy(dst_hbm.at[pl.ds(ebase + w * B, B)], dstw_v)
        pltpu.sync_copy(xl_hbm.at[srcw_v], xl_v)
        scale_and_scatter(w, HALF)

    plsc.subcore_barrier()
    drain_acc(1)
    pltpu.sync_copy(den_v, den_hbm.at[wid])
    pltpu.sync_copy(cnt_v, cnt_hbm.at[wid])


def _sc_edge(xl, xr, src_w, dst_w, att, mh16):
    f32 = jnp.float32
    mesh = plsc.VectorSubcoreMesh(core_axis_name="c", subcore_axis_name="s")
    cp = pltpu.CompilerParams()
    if "needs_layout_passes" in pltpu.CompilerParams.__dataclass_fields__:
        cp = dataclasses.replace(cp, needs_layout_passes=False)
    run = pl.kernel(
        _sc_edge_kernel,
        out_type=(
            jax.ShapeDtypeStruct((NC, 2, HALF, C), f32),
            jax.ShapeDtypeStruct((NW, DH, C), f32),
            jax.ShapeDtypeStruct((NW, DH, C), f32),
        ),
        mesh=mesh,
        scratch_types=[
            pltpu.VMEM((B,), jnp.int32),
            pltpu.VMEM((B,), jnp.int32),
            pltpu.VMEM((B, C), f32),
            pltpu.VMEM((B, C), f32),
            pltpu.VMEM((NWIN, B), f32),
            pltpu.VMEM((B,), f32),
            pltpu.VMEM((B,), jnp.int32),
            pltpu.VMEM((DH, C), f32),
            pltpu.VMEM((DH, C), f32),
            pltpu.VMEM((C,), f32),
            pltpu.VMEM((LANES,), f32),
            pltpu.VMEM((ZROWS, C), f32),
            pltpu.VMEM_SHARED((HALF, C), f32),
        ],
        compiler_params=cp,
    )
    return run(xl, xr, src_w, dst_w, att, mh16)


def _post_body(outp_ref, den_ref, cnt_ref, bias_ref, o_ref):
    acc = (outp_ref[0].reshape(2 * HALF, C)[:N]
           + outp_ref[1].reshape(2 * HALF, C)[:N])
    den = jnp.sum(den_ref[...], axis=0)[:N]
    cnt = jnp.sum(cnt_ref[...], axis=0)[:N]
    ok = jnp.logical_and(cnt > 0.0, den > 0.0)
    scale = jnp.where(ok, 1.0 / (den * jnp.maximum(cnt, 1.0)), 0.0)
    o_ref[...] = acc * scale[:, None] + bias_ref[...]


def _tc_post(outp, den, cnt, bias2d):
    return pl.pallas_call(
        _post_body,
        out_shape=jax.ShapeDtypeStruct((N, C), jnp.float32),
    )(outp, den, cnt, bias2d)


def kernel(x, edge_index, W_l, W_r, att, bias):
    xl, xr, _, _, mh = _tc_pre(x, W_l, W_r, att.reshape(1, C))
    mh16 = mh[0, :LANES]
    outp, den, cnt = _sc_edge(xl, xr, edge_index[0], edge_index[1], att, mh16)
    den2 = den.reshape(NW, NP)[:, :N]
    cnt2 = cnt.reshape(NW, NP)[:, :N]
    return _tc_post(outp, den2, cnt2, bias.reshape(1, C))


# speedup vs baseline: 12.2358x; 1.5992x over previous
import dataclasses
import functools

import jax
import jax.numpy as jnp
from jax import lax
from jax.experimental import pallas as pl
from jax.experimental.pallas import tpu as pltpu
from jax.experimental.pallas import tpu_sc as plsc

N = 10000
E = 320000
C = 128
NC = 2
NS = 16
NW = NC * NS
EPT = E // NW
B = 80
NWIN = EPT // B
LANES = 16
NP = 10240
RB = 1000

NEG_SLOPE = 0.2


def _pre_body(x_ref, wl_ref, wr_ref, att_ref, xl_ref, xr_ref, ml_ref, mr_ref,
              mh_ref):
    i = pl.program_id(0)
    x = x_ref[...]
    dn = (((1,), (0,)), ((), ()))
    xl = lax.dot_general(x, wl_ref[...], dn,
                         precision=lax.Precision.HIGHEST,
                         preferred_element_type=jnp.float32)
    xr = lax.dot_general(x, wr_ref[...], dn,
                         precision=lax.Precision.HIGHEST,
                         preferred_element_type=jnp.float32)
    xl_ref[...] = xl
    xr_ref[...] = xr
    msql = jnp.max(jnp.sum(xl * xl, axis=1))
    msqr = jnp.max(jnp.sum(xr * xr, axis=1))

    @pl.when(i == 0)
    def _():
        ml_ref[...] = jnp.zeros_like(ml_ref)
        mr_ref[...] = jnp.zeros_like(mr_ref)

    ml_ref[...] = jnp.maximum(ml_ref[...], msql)
    mr_ref[...] = jnp.maximum(mr_ref[...], msqr)

    @pl.when(i == pl.num_programs(0) - 1)
    def _():
        a = att_ref[...]
        attn = jnp.sqrt(jnp.sum(a * a))
        mh_ref[...] = attn * (jnp.sqrt(ml_ref[...]) + jnp.sqrt(mr_ref[...]))


def _tc_pre(x, W_l, W_r, att2d):
    f32 = jnp.float32
    return pl.pallas_call(
        _pre_body,
        grid=(N // RB,),
        in_specs=[
            pl.BlockSpec((RB, C), lambda i: (i, 0)),
            pl.BlockSpec((C, C), lambda i: (0, 0)),
            pl.BlockSpec((C, C), lambda i: (0, 0)),
            pl.BlockSpec((1, C), lambda i: (0, 0)),
        ],
        out_specs=[
            pl.BlockSpec((RB, C), lambda i: (i, 0)),
            pl.BlockSpec((RB, C), lambda i: (i, 0)),
            pl.BlockSpec((8, 128), lambda i: (0, 0)),
            pl.BlockSpec((8, 128), lambda i: (0, 0)),
            pl.BlockSpec((8, 128), lambda i: (0, 0)),
        ],
        out_shape=[
            jax.ShapeDtypeStruct((N, C), f32),
            jax.ShapeDtypeStruct((N, C), f32),
            jax.ShapeDtypeStruct((8, 128), f32),
            jax.ShapeDtypeStruct((8, 128), f32),
            jax.ShapeDtypeStruct((8, 128), f32),
        ],
    )(x, W_l, W_r, att2d)


HALF = 5120
TROWS = HALF // NS
ZROWS = 16
ZCHUNKS = TROWS // ZROWS
DH = NP // C


def _sc_edge_kernel(xl_hbm, xr_hbm, src_hbm, dst_hbm, att_hbm, mh_hbm,
                    outp_hbm, den_hbm, cnt_hbm,
                    srcw_v, dstw_v, xl_v, xr_v, srcw_b, dstw_b, xl_b, xr_b,
                    ex_all, exz_v, cdst_v,
                    den_v, cnt_v, att_v, mh_v, zbuf_v, acc_sh, gsem_a, gsem_b):
    c = lax.axis_index("c")
    s = lax.axis_index("s")
    wid = s * NC + c

    pltpu.sync_copy(att_hbm, att_v)
    pltpu.sync_copy(mh_hbm, mh_v)

    zero16f = jnp.zeros((LANES,), jnp.float32)

    @pl.loop(0, ZROWS)
    def _zero_zb(r):
        for k in range(C // LANES):
            zbuf_v[r, pl.ds(k * LANES, LANES)] = zero16f

    @pl.loop(0, DH)
    def _zero_dc(r):
        for k in range(C // LANES):
            den_v[r, pl.ds(k * LANES, LANES)] = zero16f
            cnt_v[r, pl.ds(k * LANES, LANES)] = zero16f

    att_chunks = [att_v[pl.ds(k * LANES, LANES)] for k in range(C // LANES)]
    mh16 = mh_v[...]
    last_lane = lax.iota(jnp.int32, LANES) == (LANES - 1)
    ones16 = jnp.ones((LANES,), jnp.float32)
    row0 = s * TROWS

    def zero_acc():
        for b in range(ZCHUNKS):
            pltpu.sync_copy(zbuf_v, acc_sh.at[pl.ds(row0 + b * ZROWS, ZROWS)])

    def drain_acc(p):
        for b in range(ZCHUNKS):
            r = row0 + b * ZROWS
            pltpu.sync_copy(acc_sh.at[pl.ds(r, ZROWS)],
                            outp_hbm.at[c, p, pl.ds(r, ZROWS)])

    def route(dst16, ex16, base):
        idxm = dst16 - base
        neg = idxm < 0
        high = idxm >= HALF
        valid = jnp.logical_and(jnp.logical_not(neg), jnp.logical_not(high))
        cidx = idxm + jnp.where(neg, HALF, 0) - jnp.where(high, HALF, 0)
        exz = jnp.where(valid, ex16, 0.0)
        return cidx, exz

    def scale_and_scatter(w, base, xlr, dstr):
        for j in range(B // LANES):
            ex16 = ex_all[w, pl.ds(j * LANES, LANES)]
            dst16 = dstr[pl.ds(j * LANES, LANES)]
            cidx, exz = route(dst16, ex16, base)
            cdst_v[pl.ds(j * LANES, LANES)] = cidx
            exz_v[pl.ds(j * LANES, LANES)] = exz

        @plsc.parallel_loop(0, B, unroll=4)
        def _scale(e):
            exb = plsc.load_gather(exz_v, [jnp.full((LANES,), e, jnp.int32)])
            for k in range(C // LANES):
                xlr[e, pl.ds(k * LANES, LANES)] = (
                    xlr[e, pl.ds(k * LANES, LANES)] * exb)

        pltpu.sync_copy(xlr, acc_sh.at[cdst_v], add=True)

    def fetch(w, srcr, dstr, xlr, xrr, sem, with_xr):
        pltpu.sync_copy(src_hbm.at[pl.ds(ebase + w * B, B)], srcr)
        pltpu.sync_copy(dst_hbm.at[pl.ds(ebase + w * B, B)], dstr)
        pltpu.async_copy(xl_hbm.at[srcr], xlr, sem)
        if with_xr:
            pltpu.async_copy(xr_hbm.at[dstr], xrr, sem)

    def wait_fetch(srcr, dstr, xlr, xrr, sem, with_xr):
        pltpu.make_async_copy(xl_hbm.at[srcr], xlr, sem).wait()
        if with_xr:
            pltpu.make_async_copy(xr_hbm.at[dstr], xrr, sem).wait()

    zero_acc()
    plsc.subcore_barrier()

    ebase = wid * EPT

    def body0(w, xlr, xrr, dstr):
        @plsc.parallel_loop(0, B, unroll=2)
        def _logit(e):
            acc = zero16f
            for k in range(C // LANES):
                a = xlr[e, pl.ds(k * LANES, LANES)]
                bb = xrr[e, pl.ds(k * LANES, LANES)]
                v = a + bb
                lr = jnp.maximum(v, NEG_SLOPE * v)
                acc = acc + lr * att_chunks[k]
            cum = plsc.cumsum(acc)
            plsc.store_scatter(ex_all,
                               [jnp.full((LANES,), w, jnp.int32),
                                jnp.full((LANES,), e, jnp.int32)],
                               cum, mask=last_lane)

        for j in range(B // LANES):
            lg = ex_all[w, pl.ds(j * LANES, LANES)]
            ex = jnp.exp(lg - mh16)
            ex_all[w, pl.ds(j * LANES, LANES)] = ex
            dst16 = dstr[pl.ds(j * LANES, LANES)]
            hi = lax.shift_right_logical(dst16, 7)
            lo = lax.bitwise_and(dst16, 127)
            plsc.addupdate_scatter(den_v, [hi, lo], ex)
            plsc.addupdate_scatter(cnt_v, [hi, lo], ones16)

        scale_and_scatter(w, 0, xlr, dstr)

    bufa = (srcw_v, dstw_v, xl_v, xr_v, gsem_a)
    bufb = (srcw_b, dstw_b, xl_b, xr_b, gsem_b)

    fetch(0, *bufa[:4], gsem_a, True)

    @pl.loop(0, NWIN - 1, step=2)
    def _window0(w):
        fetch(w + 1, *bufb[:4], gsem_b, True)
        wait_fetch(*bufa[:4], gsem_a, True)
        body0(w, xl_v, xr_v, dstw_v)
        fetch(w + 2, *bufa[:4], gsem_a, True)
        wait_fetch(*bufb[:4], gsem_b, True)
        body0(w + 1, xl_b, xr_b, dstw_b)

    wait_fetch(*bufa[:4], gsem_a, True)
    body0(NWIN - 1, xl_v, xr_v, dstw_v)

    plsc.subcore_barrier()
    drain_acc(0)
    zero_acc()
    plsc.subcore_barrier()

    fetch(0, *bufa[:4], gsem_a, False)

    @pl.loop(0, NWIN - 1, step=2)
    def _window1(w):
        fetch(w + 1, *bufb[:4], gsem_b, False)
        wait_fetch(*bufa[:4], gsem_a, False)
        scale_and_scatter(w, HALF, xl_v, dstw_v)
        fetch(w + 2, *bufa[:4], gsem_a, False)
        wait_fetch(*bufb[:4], gsem_b, False)
        scale_and_scatter(w + 1, HALF, xl_b, dstw_b)

    wait_fetch(*bufa[:4], gsem_a, False)
    scale_and_scatter(NWIN - 1, HALF, xl_v, dstw_v)

    plsc.subcore_barrier()
    drain_acc(1)
    pltpu.sync_copy(den_v, den_hbm.at[wid])
    pltpu.sync_copy(cnt_v, cnt_hbm.at[wid])


def _sc_edge(xl, xr, src_w, dst_w, att, mh16):
    f32 = jnp.float32
    mesh = plsc.VectorSubcoreMesh(core_axis_name="c", subcore_axis_name="s")
    cp = pltpu.CompilerParams()
    if "needs_layout_passes" in pltpu.CompilerParams.__dataclass_fields__:
        cp = dataclasses.replace(cp, needs_layout_passes=False)
    run = pl.kernel(
        _sc_edge_kernel,
        out_type=(
            jax.ShapeDtypeStruct((NC, 2, HALF, C), f32),
            jax.ShapeDtypeStruct((NW, DH, C), f32),
            jax.ShapeDtypeStruct((NW, DH, C), f32),
        ),
        mesh=mesh,
        scratch_types=[
            pltpu.VMEM((B,), jnp.int32),
            pltpu.VMEM((B,), jnp.int32),
            pltpu.VMEM((B, C), f32),
            pltpu.VMEM((B, C), f32),
            pltpu.VMEM((B,), jnp.int32),
            pltpu.VMEM((B,), jnp.int32),
            pltpu.VMEM((B, C), f32),
            pltpu.VMEM((B, C), f32),
            pltpu.VMEM((NWIN, B), f32),
            pltpu.VMEM((B,), f32),
            pltpu.VMEM((B,), jnp.int32),
            pltpu.VMEM((DH, C), f32),
            pltpu.VMEM((DH, C), f32),
            pltpu.VMEM((C,), f32),
            pltpu.VMEM((LANES,), f32),
            pltpu.VMEM((ZROWS, C), f32),
            pltpu.VMEM_SHARED((HALF, C), f32),
            pltpu.SemaphoreType.DMA,
            pltpu.SemaphoreType.DMA,
        ],
        compiler_params=cp,
    )
    return run(xl, xr, src_w, dst_w, att, mh16)


def _post_body(outp_ref, den_ref, cnt_ref, bias_ref, o_ref):
    acc = (outp_ref[0].reshape(2 * HALF, C)[:N]
           + outp_ref[1].reshape(2 * HALF, C)[:N])
    den = jnp.sum(den_ref[...], axis=0)[:N]
    cnt = jnp.sum(cnt_ref[...], axis=0)[:N]
    ok = jnp.logical_and(cnt > 0.0, den > 0.0)
    scale = jnp.where(ok, 1.0 / (den * jnp.maximum(cnt, 1.0)), 0.0)
    o_ref[...] = acc * scale[:, None] + bias_ref[...]


def _tc_post(outp, den, cnt, bias2d):
    return pl.pallas_call(
        _post_body,
        out_shape=jax.ShapeDtypeStruct((N, C), jnp.float32),
    )(outp, den, cnt, bias2d)


def kernel(x, edge_index, W_l, W_r, att, bias):
    xl, xr, _, _, mh = _tc_pre(x, W_l, W_r, att.reshape(1, C))
    mh16 = mh[0, :LANES]
    outp, den, cnt = _sc_edge(xl, xr, edge_index[0], edge_index[1], att, mh16)
    den2 = den.reshape(NW, NP)[:, :N]
    cnt2 = cnt.reshape(NW, NP)[:, :N]
    return _tc_post(outp, den2, cnt2, bias.reshape(1, C))


# speedup vs baseline: 14.3828x; 1.1755x over previous
import dataclasses
import functools

import jax
import jax.numpy as jnp
from jax import lax
from jax.experimental import pallas as pl
from jax.experimental.pallas import tpu as pltpu
from jax.experimental.pallas import tpu_sc as plsc

N = 10000
E = 320000
C = 128
NC = 2
NS = 16
NW = NC * NS
EPT = E // NW
B = 80
NWIN = EPT // B
LANES = 16
NP = 10240
RB = 1000

NEG_SLOPE = 0.2


def _pre_body(x_ref, wl_ref, wr_ref, att_ref, xl_ref, xr_ref, ml_ref, mr_ref,
              mh_ref):
    i = pl.program_id(0)
    x = x_ref[...]
    dn = (((1,), (0,)), ((), ()))
    xl = lax.dot_general(x, wl_ref[...], dn,
                         precision=lax.Precision.HIGHEST,
                         preferred_element_type=jnp.float32)
    xr = lax.dot_general(x, wr_ref[...], dn,
                         precision=lax.Precision.HIGHEST,
                         preferred_element_type=jnp.float32)
    xl_ref[...] = xl
    xr_ref[...] = xr
    msql = jnp.max(jnp.sum(xl * xl, axis=1))
    msqr = jnp.max(jnp.sum(xr * xr, axis=1))

    @pl.when(i == 0)
    def _():
        ml_ref[...] = jnp.zeros_like(ml_ref)
        mr_ref[...] = jnp.zeros_like(mr_ref)

    ml_ref[...] = jnp.maximum(ml_ref[...], msql)
    mr_ref[...] = jnp.maximum(mr_ref[...], msqr)

    @pl.when(i == pl.num_programs(0) - 1)
    def _():
        a = att_ref[...]
        attn = jnp.sqrt(jnp.sum(a * a))
        mh_ref[...] = attn * (jnp.sqrt(ml_ref[...]) + jnp.sqrt(mr_ref[...]))


def _tc_pre(x, W_l, W_r, att2d):
    f32 = jnp.float32
    return pl.pallas_call(
        _pre_body,
        grid=(N // RB,),
        in_specs=[
            pl.BlockSpec((RB, C), lambda i: (i, 0)),
            pl.BlockSpec((C, C), lambda i: (0, 0)),
            pl.BlockSpec((C, C), lambda i: (0, 0)),
            pl.BlockSpec((1, C), lambda i: (0, 0)),
        ],
        out_specs=[
            pl.BlockSpec((RB, C), lambda i: (i, 0)),
            pl.BlockSpec((RB, C), lambda i: (i, 0)),
            pl.BlockSpec((8, 128), lambda i: (0, 0)),
            pl.BlockSpec((8, 128), lambda i: (0, 0)),
            pl.BlockSpec((8, 128), lambda i: (0, 0)),
        ],
        out_shape=[
            jax.ShapeDtypeStruct((N, C), f32),
            jax.ShapeDtypeStruct((N, C), f32),
            jax.ShapeDtypeStruct((8, 128), f32),
            jax.ShapeDtypeStruct((8, 128), f32),
            jax.ShapeDtypeStruct((8, 128), f32),
        ],
    )(x, W_l, W_r, att2d)


HALF = 5120
TROWS = HALF // NS
ZROWS = 16
ZCHUNKS = TROWS // ZROWS
DH = NP // C


def _sc_edge_kernel(xl_hbm, xr_hbm, src_hbm, dst_hbm, att_hbm, mh_hbm,
                    outp_hbm, den_hbm, cnt_hbm,
                    srcw_v, dstw_v, xl_v, xr_v, srcw_b, dstw_b, xl_b, xr_b,
                    ex_all, exz_v, cdst_v, msg_v,
                    den_v, cnt_v, att_v, mh_v, acc_sh, gsem_a, gsem_b,
                    ssem):
    c = lax.axis_index("c")
    s = lax.axis_index("s")
    wid = s * NC + c

    pltpu.sync_copy(att_hbm, att_v)
    pltpu.sync_copy(mh_hbm, mh_v)

    zero16f = jnp.zeros((LANES,), jnp.float32)

    @pl.loop(0, DH)
    def _zero_dc(r):
        for k in range(C // LANES):
            den_v[r, pl.ds(k * LANES, LANES)] = zero16f
            cnt_v[r, pl.ds(k * LANES, LANES)] = zero16f

    att_chunks = [att_v[pl.ds(k * LANES, LANES)] for k in range(C // LANES)]
    mh16 = mh_v[...]
    last_lane = lax.iota(jnp.int32, LANES) == (LANES - 1)
    ones16 = jnp.ones((LANES,), jnp.float32)
    row0 = s * TROWS

    def zero_msg():
        @pl.loop(0, B)
        def _zm(r):
            for k in range(C // LANES):
                msg_v[r, pl.ds(k * LANES, LANES)] = zero16f

    def zero_acc():
        for b in range(ZCHUNKS):
            pltpu.sync_copy(msg_v.at[pl.ds(0, ZROWS)],
                            acc_sh.at[pl.ds(row0 + b * ZROWS, ZROWS)])

    def drain_acc(p):
        for b in range(ZCHUNKS):
            r = row0 + b * ZROWS
            pltpu.sync_copy(acc_sh.at[pl.ds(r, ZROWS)],
                            outp_hbm.at[c, p, pl.ds(r, ZROWS)])

    def route(dst16, ex16, base):
        idxm = dst16 - base
        neg = idxm < 0
        high = idxm >= HALF
        valid = jnp.logical_and(jnp.logical_not(neg), jnp.logical_not(high))
        cidx = idxm + jnp.where(neg, HALF, 0) - jnp.where(high, HALF, 0)
        exz = jnp.where(valid, ex16, 0.0)
        return cidx, exz

    def wait_scatter():
        pltpu.make_async_copy(msg_v, acc_sh.at[cdst_v], ssem).wait()

    def prime_scatter():
        for j in range(B // LANES):
            cdst_v[pl.ds(j * LANES, LANES)] = (
                lax.iota(jnp.int32, LANES) + j * LANES)

        pltpu.async_copy(msg_v, acc_sh.at[cdst_v], ssem, add=True)

    def scale_and_scatter(w, base, xlr, dstr):
        wait_scatter()
        for j in range(B // LANES):
            ex16 = ex_all[w, pl.ds(j * LANES, LANES)]
            dst16 = dstr[pl.ds(j * LANES, LANES)]
            cidx, exz = route(dst16, ex16, base)
            cdst_v[pl.ds(j * LANES, LANES)] = cidx
            exz_v[pl.ds(j * LANES, LANES)] = exz

        @plsc.parallel_loop(0, B, unroll=4)
        def _scale(e):
            exb = plsc.load_gather(exz_v, [jnp.full((LANES,), e, jnp.int32)])
            for k in range(C // LANES):
                msg_v[e, pl.ds(k * LANES, LANES)] = (
                    xlr[e, pl.ds(k * LANES, LANES)] * exb)

        pltpu.async_copy(msg_v, acc_sh.at[cdst_v], ssem, add=True)

    def fetch(w, srcr, dstr, xlr, xrr, sem, with_xr):
        pltpu.sync_copy(src_hbm.at[pl.ds(ebase + w * B, B)], srcr)
        pltpu.sync_copy(dst_hbm.at[pl.ds(ebase + w * B, B)], dstr)
        pltpu.async_copy(xl_hbm.at[srcr], xlr, sem)
        if with_xr:
            pltpu.async_copy(xr_hbm.at[dstr], xrr, sem)

    def wait_fetch(srcr, dstr, xlr, xrr, sem, with_xr):
        pltpu.make_async_copy(xl_hbm.at[srcr], xlr, sem).wait()
        if with_xr:
            pltpu.make_async_copy(xr_hbm.at[dstr], xrr, sem).wait()

    zero_msg()
    zero_acc()
    plsc.subcore_barrier()

    ebase = wid * EPT

    def body0(w, xlr, xrr, dstr):
        @plsc.parallel_loop(0, B, unroll=2)
        def _logit(e):
            acc = zero16f
            for k in range(C // LANES):
                a = xlr[e, pl.ds(k * LANES, LANES)]
                bb = xrr[e, pl.ds(k * LANES, LANES)]
                v = a + bb
                lr = jnp.maximum(v, NEG_SLOPE * v)
                acc = acc + lr * att_chunks[k]
            cum = plsc.cumsum(acc)
            plsc.store_scatter(ex_all,
                               [jnp.full((LANES,), w, jnp.int32),
                                jnp.full((LANES,), e, jnp.int32)],
                               cum, mask=last_lane)

        for j in range(B // LANES):
            lg = ex_all[w, pl.ds(j * LANES, LANES)]
            ex = jnp.exp(lg - mh16)
            ex_all[w, pl.ds(j * LANES, LANES)] = ex
            dst16 = dstr[pl.ds(j * LANES, LANES)]
            hi = lax.shift_right_logical(dst16, 7)
            lo = lax.bitwise_and(dst16, 127)
            plsc.addupdate_scatter(den_v, [hi, lo], ex)
            plsc.addupdate_scatter(cnt_v, [hi, lo], ones16)

        scale_and_scatter(w, 0, xlr, dstr)

    bufa = (srcw_v, dstw_v, xl_v, xr_v, gsem_a)
    bufb = (srcw_b, dstw_b, xl_b, xr_b, gsem_b)

    prime_scatter()
    fetch(0, *bufa[:4], gsem_a, True)

    @pl.loop(0, NWIN - 1, step=2)
    def _window0(w):
        fetch(w + 1, *bufb[:4], gsem_b, True)
        wait_fetch(*bufa[:4], gsem_a, True)
        body0(w, xl_v, xr_v, dstw_v)
        fetch(w + 2, *bufa[:4], gsem_a, True)
        wait_fetch(*bufb[:4], gsem_b, True)
        body0(w + 1, xl_b, xr_b, dstw_b)

    wait_fetch(*bufa[:4], gsem_a, True)
    body0(NWIN - 1, xl_v, xr_v, dstw_v)
    wait_scatter()

    plsc.subcore_barrier()
    drain_acc(0)
    zero_msg()
    zero_acc()
    plsc.subcore_barrier()

    prime_scatter()
    fetch(0, *bufa[:4], gsem_a, False)

    @pl.loop(0, NWIN - 1, step=2)
    def _window1(w):
        fetch(w + 1, *bufb[:4], gsem_b, False)
        wait_fetch(*bufa[:4], gsem_a, False)
        scale_and_scatter(w, HALF, xl_v, dstw_v)
        fetch(w + 2, *bufa[:4], gsem_a, False)
        wait_fetch(*bufb[:4], gsem_b, False)
        scale_and_scatter(w + 1, HALF, xl_b, dstw_b)

    wait_fetch(*bufa[:4], gsem_a, False)
    scale_and_scatter(NWIN - 1, HALF, xl_v, dstw_v)
    wait_scatter()

    plsc.subcore_barrier()
    drain_acc(1)
    pltpu.sync_copy(den_v, den_hbm.at[wid])
    pltpu.sync_copy(cnt_v, cnt_hbm.at[wid])


def _sc_edge(xl, xr, src_w, dst_w, att, mh16):
    f32 = jnp.float32
    mesh = plsc.VectorSubcoreMesh(core_axis_name="c", subcore_axis_name="s")
    cp = pltpu.CompilerParams()
    if "needs_layout_passes" in pltpu.CompilerParams.__dataclass_fields__:
        cp = dataclasses.replace(cp, needs_layout_passes=False)
    run = pl.kernel(
        _sc_edge_kernel,
        out_type=(
            jax.ShapeDtypeStruct((NC, 2, HALF, C), f32),
            jax.ShapeDtypeStruct((NW, DH, C), f32),
            jax.ShapeDtypeStruct((NW, DH, C), f32),
        ),
        mesh=mesh,
        scratch_types=[
            pltpu.VMEM((B,), jnp.int32),
            pltpu.VMEM((B,), jnp.int32),
            pltpu.VMEM((B, C), f32),
            pltpu.VMEM((B, C), f32),
            pltpu.VMEM((B,), jnp.int32),
            pltpu.VMEM((B,), jnp.int32),
            pltpu.VMEM((B, C), f32),
            pltpu.VMEM((B, C), f32),
            pltpu.VMEM((NWIN, B), f32),
            pltpu.VMEM((B,), f32),
            pltpu.VMEM((B,), jnp.int32),
            pltpu.VMEM((B, C), f32),
            pltpu.VMEM((DH, C), f32),
            pltpu.VMEM((DH, C), f32),
            pltpu.VMEM((C,), f32),
            pltpu.VMEM((LANES,), f32),
            pltpu.VMEM_SHARED((HALF, C), f32),
            pltpu.SemaphoreType.DMA,
            pltpu.SemaphoreType.DMA,
            pltpu.SemaphoreType.DMA,
        ],
        compiler_params=cp,
    )
    return run(xl, xr, src_w, dst_w, att, mh16)


def _post_body(outp_ref, den_ref, cnt_ref, bias_ref, o_ref):
    acc = (outp_ref[0].reshape(2 * HALF, C)[:N]
           + outp_ref[1].reshape(2 * HALF, C)[:N])
    den = jnp.sum(den_ref[...], axis=0)[:N]
    cnt = jnp.sum(cnt_ref[...], axis=0)[:N]
    ok = jnp.logical_and(cnt > 0.0, den > 0.0)
    scale = jnp.where(ok, 1.0 / (den * jnp.maximum(cnt, 1.0)), 0.0)
    o_ref[...] = acc * scale[:, None] + bias_ref[...]


def _tc_post(outp, den, cnt, bias2d):
    return pl.pallas_call(
        _post_body,
        out_shape=jax.ShapeDtypeStruct((N, C), jnp.float32),
    )(outp, den, cnt, bias2d)


def kernel(x, edge_index, W_l, W_r, att, bias):
    xl, xr, _, _, mh = _tc_pre(x, W_l, W_r, att.reshape(1, C))
    mh16 = mh[0, :LANES]
    outp, den, cnt = _sc_edge(xl, xr, edge_index[0], edge_index[1], att, mh16)
    den2 = den.reshape(NW, NP)[:, :N]
    cnt2 = cnt.reshape(NW, NP)[:, :N]
    return _tc_post(outp, den2, cnt2, bias.reshape(1, C))
